# baseline (speedup 1.0000x reference)
import jax
import jax.numpy as jnp
from jax import lax
from jax.experimental import pallas as pl
from jax.experimental.pallas import tpu as pltpu, tpu_sc as plsc
import functools

NC = 2
NS = 16
NW = NC * NS
B_TOTAL = 16384
DIM = 64
NUM_ENTS_C = 1000000
NUM_RELS_C = 1000
BPW = B_TOTAL // NW
CHUNK = 128
NCHUNK = BPW // CHUNK
GROUPS = BPW // 16
BLK = 65536


def _proj_body(wt_ref, tab_ref, rel_ref, b_ref, o0_ref, o1_ref,
               r0_ref, r1_ref):
    res = jnp.dot(wt_ref[...], tab_ref[...],
                  preferred_element_type=jnp.float32)
    o0_ref[...] = res[0]
    o1_ref[...] = res[1]

    @pl.when(pl.program_id(0) == 0)
    def _():
        rp = jnp.dot(wt_ref[...], rel_ref[...],
                     preferred_element_type=jnp.float32) + b_ref[...]
        r0_ref[...] = rp[0]
        r1_ref[...] = rp[1]


def _sc_body(e0_h, e1_h, r0_h, r1_h, head_h, rel_h, tail_h, o0_h, o1_h,
             idx_h, idx_r, idx_t, hp0, hp1, tp0, tp1, rp0, rp1,
             ov0, ov1, sem):
    c = lax.axis_index("c")
    s = lax.axis_index("s")
    wid = s * NC + c
    base = wid * BPW

    pltpu.sync_copy(head_h.at[wid], idx_h)
    pltpu.sync_copy(rel_h.at[wid], idx_r)
    pltpu.sync_copy(tail_h.at[wid], idx_t)

    cps = []
    for k in range(NCHUNK):
        sl = pl.ds(k * CHUNK, CHUNK)
        cps.append(pltpu.async_copy(e0_h.at[idx_h.at[k]], hp0.at[sl], sem))
        cps.append(pltpu.async_copy(e1_h.at[idx_h.at[k]], hp1.at[sl], sem))
        cps.append(pltpu.async_copy(e0_h.at[idx_t.at[k]], tp0.at[sl], sem))
        cps.append(pltpu.async_copy(e1_h.at[idx_t.at[k]], tp1.at[sl], sem))
        cps.append(pltpu.async_copy(r0_h.at[idx_r.at[k]], rp0.at[sl], sem))
        cps.append(pltpu.async_copy(r1_h.at[idx_r.at[k]], rp1.at[sl], sem))
    for cp in cps:
        cp.wait()

    def group_step(g, carry):
        sl = pl.ds(pl.multiple_of(g * 16, 16), 16)
        a0 = hp0[sl] + rp0[sl] - tp0[sl]
        a1 = hp1[sl] + rp1[sl] - tp1[sl]
        ov0[sl] = 1.0 / (1.0 + jnp.exp(-a0))
        ov1[sl] = 1.0 / (1.0 + jnp.exp(-a1))
        return carry

    lax.fori_loop(0, GROUPS, group_step, 0)
    pltpu.sync_copy(ov0, o0_h.at[pl.ds(base, BPW)])
    pltpu.sync_copy(ov1, o1_h.at[pl.ds(base, BPW)])


@jax.jit
def _run(head3, rel3, tail3, ent_emb, rel_emb, lin_W, lin_b):
    wt = lin_W.T
    ent_t = ent_emb.T
    rel_t = rel_emb.T
    b2 = lin_b.reshape(2, 1)

    grid = (NUM_ENTS_C + BLK - 1) // BLK
    e0, e1, r0, r1 = pl.pallas_call(
        _proj_body,
        grid=(grid,),
        in_specs=[
            pl.BlockSpec((2, DIM), lambda i: (0, 0)),
            pl.BlockSpec((DIM, BLK), lambda i: (0, i)),
            pl.BlockSpec((DIM, NUM_RELS_C), lambda i: (0, 0)),
            pl.BlockSpec((2, 1), lambda i: (0, 0)),
        ],
        out_specs=[
            pl.BlockSpec((BLK,), lambda i: (i,)),
            pl.BlockSpec((BLK,), lambda i: (i,)),
            pl.BlockSpec((NUM_RELS_C,), lambda i: (0,)),
            pl.BlockSpec((NUM_RELS_C,), lambda i: (0,)),
        ],
        out_shape=[
            jax.ShapeDtypeStruct((NUM_ENTS_C,), jnp.float32),
            jax.ShapeDtypeStruct((NUM_ENTS_C,), jnp.float32),
            jax.ShapeDtypeStruct((NUM_RELS_C,), jnp.float32),
            jax.ShapeDtypeStruct((NUM_RELS_C,), jnp.float32),
        ],
        compiler_params=pltpu.CompilerParams(vmem_limit_bytes=60000000),
    )(wt, ent_t, rel_t, b2)

    mesh = plsc.VectorSubcoreMesh(core_axis_name="c", subcore_axis_name="s")
    f = pl.kernel(
        _sc_body,
        out_type=(
            jax.ShapeDtypeStruct((B_TOTAL,), jnp.float32),
            jax.ShapeDtypeStruct((B_TOTAL,), jnp.float32),
        ),
        mesh=mesh,
        compiler_params=pltpu.CompilerParams(needs_layout_passes=False),
        scratch_types=[
            pltpu.VMEM((NCHUNK, CHUNK), jnp.int32),
            pltpu.VMEM((NCHUNK, CHUNK), jnp.int32),
            pltpu.VMEM((NCHUNK, CHUNK), jnp.int32),
            pltpu.VMEM((BPW,), jnp.float32),
            pltpu.VMEM((BPW,), jnp.float32),
            pltpu.VMEM((BPW,), jnp.float32),
            pltpu.VMEM((BPW,), jnp.float32),
            pltpu.VMEM((BPW,), jnp.float32),
            pltpu.VMEM((BPW,), jnp.float32),
            pltpu.VMEM((BPW,), jnp.float32),
            pltpu.VMEM((BPW,), jnp.float32),
            pltpu.SemaphoreType.DMA,
        ],
        name="transe_sc",
    )
    o0, o1 = f(e0, e1, r0, r1, head3, rel3, tail3)
    return jnp.stack([o0, o1], axis=1)


def kernel(head, rel, tail, ent_emb, rel_emb, lin_W, lin_b):
    head3 = head.astype(jnp.int32).reshape(NW, NCHUNK, CHUNK)
    rel3 = rel.astype(jnp.int32).reshape(NW, NCHUNK, CHUNK)
    tail3 = tail.astype(jnp.int32).reshape(NW, NCHUNK, CHUNK)
    return _run(head3, rel3, tail3, ent_emb, rel_emb, lin_W, lin_b)


# speedup vs baseline: 1.0286x; 1.0286x over previous
import jax
import jax.numpy as jnp
from jax import lax
from jax.experimental import pallas as pl
from jax.experimental.pallas import tpu as pltpu, tpu_sc as plsc
import functools

NC = 2
NS = 16
NW = NC * NS
B_TOTAL = 16384
DIM = 64
NUM_ENTS_C = 1000000
NUM_RELS_C = 1000
BPW = B_TOTAL // NW
CHUNK = 128
NCHUNK = BPW // CHUNK
GROUPS = BPW // 16
BLK = 32768


def _proj_body(wt_ref, tab_ref, rel_ref, b_ref, o0_ref, o1_ref,
               r0_ref, r1_ref):
    res = jnp.dot(wt_ref[...], tab_ref[...],
                  preferred_element_type=jnp.float32)
    o0_ref[...] = res[0]
    o1_ref[...] = res[1]

    @pl.when(pl.program_id(0) == 0)
    def _():
        rp = jnp.dot(wt_ref[...], rel_ref[...],
                     preferred_element_type=jnp.float32) + b_ref[...]
        r0_ref[...] = rp[0]
        r1_ref[...] = rp[1]


def _sc_body(e0_h, e1_h, r0_h, r1_h, head_h, rel_h, tail_h, o0_h, o1_h,
             idx_h, idx_r, idx_t, hp0, hp1, tp0, tp1, rp0, rp1,
             ov0, ov1, sem):
    c = lax.axis_index("c")
    s = lax.axis_index("s")
    wid = s * NC + c
    base = wid * BPW

    pltpu.sync_copy(head_h.at[wid], idx_h)
    pltpu.sync_copy(rel_h.at[wid], idx_r)
    pltpu.sync_copy(tail_h.at[wid], idx_t)

    cps = []
    for k in range(NCHUNK):
        sl = pl.ds(k * CHUNK, CHUNK)
        cps.append(pltpu.async_copy(e0_h.at[idx_h.at[k]], hp0.at[sl], sem))
        cps.append(pltpu.async_copy(e1_h.at[idx_h.at[k]], hp1.at[sl], sem))
        cps.append(pltpu.async_copy(e0_h.at[idx_t.at[k]], tp0.at[sl], sem))
        cps.append(pltpu.async_copy(e1_h.at[idx_t.at[k]], tp1.at[sl], sem))
        cps.append(pltpu.async_copy(r0_h.at[idx_r.at[k]], rp0.at[sl], sem))
        cps.append(pltpu.async_copy(r1_h.at[idx_r.at[k]], rp1.at[sl], sem))
    for cp in cps:
        cp.wait()

    def group_step(g, carry):
        sl = pl.ds(pl.multiple_of(g * 16, 16), 16)
        a0 = hp0[sl] + rp0[sl] - tp0[sl]
        a1 = hp1[sl] + rp1[sl] - tp1[sl]
        ov0[sl] = 1.0 / (1.0 + jnp.exp(-a0))
        ov1[sl] = 1.0 / (1.0 + jnp.exp(-a1))
        return carry

    lax.fori_loop(0, GROUPS, group_step, 0)
    pltpu.sync_copy(ov0, o0_h.at[pl.ds(base, BPW)])
    pltpu.sync_copy(ov1, o1_h.at[pl.ds(base, BPW)])


@jax.jit
def _run(head3, rel3, tail3, ent_emb, rel_emb, lin_W, lin_b):
    wt = lin_W.T
    ent_t = ent_emb.T
    rel_t = rel_emb.T
    b2 = lin_b.reshape(2, 1)

    grid = (NUM_ENTS_C + BLK - 1) // BLK
    e0, e1, r0, r1 = pl.pallas_call(
        _proj_body,
        grid=(grid,),
        in_specs=[
            pl.BlockSpec((2, DIM), lambda i: (0, 0)),
            pl.BlockSpec((DIM, BLK), lambda i: (0, i)),
            pl.BlockSpec((DIM, NUM_RELS_C), lambda i: (0, 0)),
            pl.BlockSpec((2, 1), lambda i: (0, 0)),
        ],
        out_specs=[
            pl.BlockSpec((BLK,), lambda i: (i,)),
            pl.BlockSpec((BLK,), lambda i: (i,)),
            pl.BlockSpec((NUM_RELS_C,), lambda i: (0,)),
            pl.BlockSpec((NUM_RELS_C,), lambda i: (0,)),
        ],
        out_shape=[
            jax.ShapeDtypeStruct((NUM_ENTS_C,), jnp.float32),
            jax.ShapeDtypeStruct((NUM_ENTS_C,), jnp.float32),
            jax.ShapeDtypeStruct((NUM_RELS_C,), jnp.float32),
            jax.ShapeDtypeStruct((NUM_RELS_C,), jnp.float32),
        ],
        compiler_params=pltpu.CompilerParams(vmem_limit_bytes=60000000),
    )(wt, ent_t, rel_t, b2)

    mesh = plsc.VectorSubcoreMesh(core_axis_name="c", subcore_axis_name="s")
    f = pl.kernel(
        _sc_body,
        out_type=(
            jax.ShapeDtypeStruct((B_TOTAL,), jnp.float32),
            jax.ShapeDtypeStruct((B_TOTAL,), jnp.float32),
        ),
        mesh=mesh,
        compiler_params=pltpu.CompilerParams(needs_layout_passes=False),
        scratch_types=[
            pltpu.VMEM((NCHUNK, CHUNK), jnp.int32),
            pltpu.VMEM((NCHUNK, CHUNK), jnp.int32),
            pltpu.VMEM((NCHUNK, CHUNK), jnp.int32),
            pltpu.VMEM((BPW,), jnp.float32),
            pltpu.VMEM((BPW,), jnp.float32),
            pltpu.VMEM((BPW,), jnp.float32),
            pltpu.VMEM((BPW,), jnp.float32),
            pltpu.VMEM((BPW,), jnp.float32),
            pltpu.VMEM((BPW,), jnp.float32),
            pltpu.VMEM((BPW,), jnp.float32),
            pltpu.VMEM((BPW,), jnp.float32),
            pltpu.SemaphoreType.DMA,
        ],
        name="transe_sc",
    )
    o0, o1 = f(e0, e1, r0, r1, head3, rel3, tail3)
    return jnp.stack([o0, o1], axis=1)


def kernel(head, rel, tail, ent_emb, rel_emb, lin_W, lin_b):
    head3 = head.astype(jnp.int32).reshape(NW, NCHUNK, CHUNK)
    rel3 = rel.astype(jnp.int32).reshape(NW, NCHUNK, CHUNK)
    tail3 = tail.astype(jnp.int32).reshape(NW, NCHUNK, CHUNK)
    return _run(head3, rel3, tail3, ent_emb, rel_emb, lin_W, lin_b)
